# baseline (speedup 1.0000x reference)
import jax
import jax.numpy as jnp
from jax.experimental import pallas as pl
from jax.experimental.pallas import tpu as pltpu

B, S, D = 4, 4096, 1024
N = B * S
ROWS = 256


def _body(m_ref, x_ref, a_ref, o_ref):
    m = m_ref[...]
    o_ref[...] = jnp.where(m, x_ref[...] + a_ref[...], x_ref[...])


def kernel(x, attack, attack_mask):
    xr = x.reshape(N, D)
    ar = attack.reshape(N, D)
    mf = attack_mask.reshape(N, 1)
    grid = (N // ROWS,)
    out = pl.pallas_call(
        _body,
        grid=grid,
        in_specs=[
            pl.BlockSpec((ROWS, 1), lambda i: (i, 0)),
            pl.BlockSpec((ROWS, D), lambda i: (i, 0)),
            pl.BlockSpec((ROWS, D), lambda i: (i, 0)),
        ],
        out_specs=pl.BlockSpec((ROWS, D), lambda i: (i, 0)),
        out_shape=jax.ShapeDtypeStruct((N, D), jnp.float32),
    )(mf, xr, ar)
    return out.reshape(B, S, D)


# speedup vs baseline: 1.1991x; 1.1991x over previous
import jax
import jax.numpy as jnp
from jax.experimental import pallas as pl
from jax.experimental.pallas import tpu as pltpu

B, S, D = 4, 4096, 1024
N = B * S
ROWS = 1024


def _body(m_ref, x_ref, a_ref, o_ref):
    m = m_ref[...]
    o_ref[...] = jnp.where(m, x_ref[...] + a_ref[...], x_ref[...])


def kernel(x, attack, attack_mask):
    xr = x.reshape(N, D)
    ar = attack.reshape(N, D)
    mf = attack_mask.reshape(N, 1)
    grid = (N // ROWS,)
    out = pl.pallas_call(
        _body,
        grid=grid,
        in_specs=[
            pl.BlockSpec((ROWS, 1), lambda i: (i, 0)),
            pl.BlockSpec((ROWS, D), lambda i: (i, 0)),
            pl.BlockSpec((ROWS, D), lambda i: (i, 0)),
        ],
        out_specs=pl.BlockSpec((ROWS, D), lambda i: (i, 0)),
        out_shape=jax.ShapeDtypeStruct((N, D), jnp.float32),
    )(mf, xr, ar)
    return out.reshape(B, S, D)
